# Initial kernel scaffold; baseline (speedup 1.0000x reference)
import jax
import jax.numpy as jnp
from jax.experimental import pallas as pl


def kernel(x, table):
    raise NotImplementedError("write your pallas kernel here")


# speedup vs baseline: 1.0431x; 1.0431x over previous
import functools
import math

import jax
import jax.numpy as jnp
from jax import lax
from jax.experimental import pallas as pl
from jax.experimental.pallas import tpu as pltpu
from jax.experimental.pallas import tpu_sc as plsc

D_MODEL = 128
SCALE = math.sqrt(D_MODEL)

_info = plsc.get_sparse_core_info()
NC, NS, L = _info.num_cores, _info.num_subcores, _info.num_lanes
NW = NC * NS

B_TOTAL = 1024 * 200
B_PER_W = B_TOTAL // NW
CHUNK = 128
N_CHUNKS = B_PER_W // CHUNK


def _emb_kernel(table_hbm, x_hbm, out_hbm, idx_v, rows_v, sem):
    wid = lax.axis_index("s") * NC + lax.axis_index("c")
    pltpu.sync_copy(x_hbm.at[wid], idx_v)

    def chunk_body(g, carry):
        pltpu.async_copy(table_hbm.at[idx_v.at[g]], rows_v, sem).wait()

        def row_body(r, c2):
            for cseg in range(D_MODEL // L):
                sl = pl.ds(cseg * L, L)
                rows_v[r, sl] = rows_v[r, sl] * SCALE
            return c2

        lax.fori_loop(0, CHUNK, row_body, 0, unroll=2)

        pltpu.sync_copy(rows_v, out_hbm.at[pl.ds(wid * B_PER_W + g * CHUNK, CHUNK)])
        return carry

    lax.fori_loop(0, N_CHUNKS, chunk_body, 0)


@functools.partial(
    pl.kernel,
    out_type=jax.ShapeDtypeStruct((B_TOTAL, D_MODEL), jnp.float32),
    mesh=plsc.VectorSubcoreMesh(core_axis_name="c", subcore_axis_name="s"),
    scratch_types=[
        pltpu.VMEM((N_CHUNKS, CHUNK), jnp.int32),
        pltpu.VMEM((CHUNK, D_MODEL), jnp.float32),
        pltpu.SemaphoreType.DMA,
    ],
)
def _emb_call(table_hbm, x_hbm, out_hbm, idx_v, rows_v, sem):
    _emb_kernel(table_hbm, x_hbm, out_hbm, idx_v, rows_v, sem)


def kernel(x, table):
    xs, ss = x.shape, table.shape
    x_flat = x.astype(jnp.int32).reshape(NW, N_CHUNKS, CHUNK)
    out = _emb_call(table, x_flat)
    return out.reshape(xs[0], xs[1], D_MODEL)


# speedup vs baseline: 1.5407x; 1.4770x over previous
import functools
import math

import jax
import jax.numpy as jnp
from jax import lax
from jax.experimental import pallas as pl
from jax.experimental.pallas import tpu as pltpu
from jax.experimental.pallas import tpu_sc as plsc

D_MODEL = 128
SCALE = math.sqrt(D_MODEL)

_info = plsc.get_sparse_core_info()
NC, NS, L = _info.num_cores, _info.num_subcores, _info.num_lanes
NW = NC * NS

B_TOTAL = 1024 * 200
B_PER_W = B_TOTAL // NW
CHUNK = 128
N_CHUNKS = B_PER_W // CHUNK


def _emb_kernel(table_hbm, x_hbm, out_hbm, idx_v, rows0, rows1, sg0, sg1, sw0, sw1):
    wid = lax.axis_index("s") * NC + lax.axis_index("c")
    base = wid * B_PER_W
    bufs = (rows0, rows1)
    gsems = (sg0, sg1)
    wsems = (sw0, sw1)

    pltpu.sync_copy(x_hbm.at[wid], idx_v)

    def gather_start(g, b):
        pltpu.async_copy(table_hbm.at[idx_v.at[g]], bufs[b], gsems[b])

    def gather_wait(g, b):
        pltpu.make_async_copy(table_hbm.at[idx_v.at[g]], bufs[b], gsems[b]).wait()

    def write_start(g, b):
        pltpu.async_copy(bufs[b], out_hbm.at[pl.ds(base + g * CHUNK, CHUNK)], wsems[b])

    def write_wait(b):
        pltpu.make_async_copy(bufs[b], out_hbm.at[pl.ds(base, CHUNK)], wsems[b]).wait()

    gather_start(0, 0)

    def outer(i, carry):
        g0 = i * 2
        for b in range(2):
            g = g0 + b
            bb = 1 - b
            gather_wait(g, b)

            @pl.when(g >= 1)
            def _():
                write_wait(bb)

            @pl.when(g + 1 < N_CHUNKS)
            def _():
                gather_start(g + 1, bb)

            def row_body(r, c2):
                for cseg in range(D_MODEL // L):
                    sl = pl.ds(cseg * L, L)
                    bufs[b][r, sl] = bufs[b][r, sl] * SCALE
                return c2

            lax.fori_loop(0, CHUNK, row_body, 0, unroll=2)
            write_start(g, b)
        return carry

    lax.fori_loop(0, N_CHUNKS // 2, outer, 0)
    write_wait((N_CHUNKS - 1) % 2)


@functools.partial(
    pl.kernel,
    out_type=jax.ShapeDtypeStruct((B_TOTAL, D_MODEL), jnp.float32),
    mesh=plsc.VectorSubcoreMesh(core_axis_name="c", subcore_axis_name="s"),
    scratch_types=[
        pltpu.VMEM((N_CHUNKS, CHUNK), jnp.int32),
        pltpu.VMEM((CHUNK, D_MODEL), jnp.float32),
        pltpu.VMEM((CHUNK, D_MODEL), jnp.float32),
        pltpu.SemaphoreType.DMA,
        pltpu.SemaphoreType.DMA,
        pltpu.SemaphoreType.DMA,
        pltpu.SemaphoreType.DMA,
    ],
)
def _emb_call(table_hbm, x_hbm, out_hbm, idx_v, rows0, rows1, sg0, sg1, sw0, sw1):
    _emb_kernel(table_hbm, x_hbm, out_hbm, idx_v, rows0, rows1, sg0, sg1, sw0, sw1)


def kernel(x, table):
    xs = x.shape
    x_flat = x.astype(jnp.int32).reshape(NW, N_CHUNKS, CHUNK)
    out = _emb_call(table, x_flat)
    return out.reshape(xs[0], xs[1], D_MODEL)


# speedup vs baseline: 1.7233x; 1.1185x over previous
import functools
import math

import jax
import jax.numpy as jnp
from jax import lax
from jax.experimental import pallas as pl
from jax.experimental.pallas import tpu as pltpu
from jax.experimental.pallas import tpu_sc as plsc

D_MODEL = 128
SCALE = math.sqrt(D_MODEL)

_info = plsc.get_sparse_core_info()
NC, NS, L = _info.num_cores, _info.num_subcores, _info.num_lanes
NW = NC * NS

B_TOTAL = 1024 * 200
B_PER_W = B_TOTAL // NW
CHUNK = 128
N_CHUNKS = B_PER_W // CHUNK
NBUF = 5


def _emb_kernel(table_hbm, x_hbm, out_hbm, idx_v, rows_v, *sems):
    gsems = sems[:NBUF]
    wsems = sems[NBUF:]
    wid = lax.axis_index("s") * NC + lax.axis_index("c")
    base = wid * B_PER_W

    pltpu.sync_copy(x_hbm.at[wid], idx_v)

    def gather_start(g, b):
        pltpu.async_copy(table_hbm.at[idx_v.at[g]], rows_v.at[b], gsems[b])

    def gather_wait(g, b):
        pltpu.make_async_copy(table_hbm.at[idx_v.at[g]], rows_v.at[b], gsems[b]).wait()

    def write_start(g, b):
        pltpu.async_copy(rows_v.at[b], out_hbm.at[pl.ds(base + g * CHUNK, CHUNK)], wsems[b])

    def write_wait(b):
        pltpu.make_async_copy(rows_v.at[b], out_hbm.at[pl.ds(base, CHUNK)], wsems[b]).wait()

    for b in range(NBUF - 1):
        gather_start(b, b)

    def outer(i, carry):
        g0 = i * NBUF
        for b in range(NBUF):
            g = g0 + b
            bprev = (b - 1) % NBUF
            gather_wait(g, b)

            @pl.when(g >= 1)
            def _():
                write_wait(bprev)

            @pl.when(g + NBUF - 1 < N_CHUNKS)
            def _():
                gather_start(g + NBUF - 1, bprev)

            def row_body(r, c2):
                for cseg in range(D_MODEL // L):
                    sl = pl.ds(cseg * L, L)
                    rows_v[b, r, sl] = rows_v[b, r, sl] * SCALE
                return c2

            lax.fori_loop(0, CHUNK, row_body, 0, unroll=2)
            write_start(g, b)
        return carry

    lax.fori_loop(0, N_CHUNKS // NBUF, outer, 0)
    write_wait((N_CHUNKS - 1) % NBUF)


@functools.partial(
    pl.kernel,
    out_type=jax.ShapeDtypeStruct((B_TOTAL, D_MODEL), jnp.float32),
    mesh=plsc.VectorSubcoreMesh(core_axis_name="c", subcore_axis_name="s"),
    scratch_types=[
        pltpu.VMEM((N_CHUNKS, CHUNK), jnp.int32),
        pltpu.VMEM((NBUF, CHUNK, D_MODEL), jnp.float32),
    ] + [pltpu.SemaphoreType.DMA] * (2 * NBUF),
)
def _emb_call(table_hbm, x_hbm, out_hbm, idx_v, rows_v, *sems):
    _emb_kernel(table_hbm, x_hbm, out_hbm, idx_v, rows_v, *sems)


def kernel(x, table):
    xs = x.shape
    x_flat = x.astype(jnp.int32).reshape(NW, N_CHUNKS, CHUNK)
    out = _emb_call(table, x_flat)
    return out.reshape(xs[0], xs[1], D_MODEL)


# speedup vs baseline: 1.7736x; 1.0292x over previous
import functools
import math

import jax
import jax.numpy as jnp
from jax import lax
from jax.experimental import pallas as pl
from jax.experimental.pallas import tpu as pltpu
from jax.experimental.pallas import tpu_sc as plsc

D_MODEL = 128
SCALE = math.sqrt(D_MODEL)

_info = plsc.get_sparse_core_info()
NC, NS, L = _info.num_cores, _info.num_subcores, _info.num_lanes
NW = NC * NS

B_TOTAL = 1024 * 200
B_PER_W = B_TOTAL // NW
CHUNK = 128
N_CHUNKS = B_PER_W // CHUNK
NBUF = 5


def _emb_kernel(table_hbm, x_hbm, out_hbm, idx_v, rows_v, *sems):
    gsems = sems[:NBUF]
    wsems = sems[NBUF:]
    wid = lax.axis_index("s") * NC + lax.axis_index("c")
    base = wid * B_PER_W

    pltpu.sync_copy(x_hbm.at[wid], idx_v)

    def gather_start(g, b):
        pltpu.async_copy(table_hbm.at[idx_v.at[g]], rows_v.at[b], gsems[b])

    def gather_wait(g, b):
        pltpu.make_async_copy(table_hbm.at[idx_v.at[g]], rows_v.at[b], gsems[b]).wait()

    def write_start(g, b):
        pltpu.async_copy(rows_v.at[b], out_hbm.at[pl.ds(base + g * CHUNK, CHUNK)], wsems[b])

    def write_wait(b):
        pltpu.make_async_copy(rows_v.at[b], out_hbm.at[pl.ds(base, CHUNK)], wsems[b]).wait()

    for b in range(NBUF - 2):
        gather_start(b, b)

    def outer(i, carry):
        g0 = i * NBUF
        for b in range(NBUF):
            g = g0 + b
            bprev2 = (b - 2) % NBUF
            gather_wait(g, b)

            @pl.when(g >= 2)
            def _():
                write_wait(bprev2)

            @pl.when(g + NBUF - 2 < N_CHUNKS)
            def _():
                gather_start(g + NBUF - 2, bprev2)

            def row_body(r, c2):
                for cseg in range(D_MODEL // L):
                    sl = pl.ds(cseg * L, L)
                    rows_v[b, r, sl] = rows_v[b, r, sl] * SCALE
                return c2

            lax.fori_loop(0, CHUNK, row_body, 0, unroll=2)
            write_start(g, b)
        return carry

    lax.fori_loop(0, N_CHUNKS // NBUF, outer, 0)
    write_wait((N_CHUNKS - 2) % NBUF)
    write_wait((N_CHUNKS - 1) % NBUF)


@functools.partial(
    pl.kernel,
    out_type=jax.ShapeDtypeStruct((B_TOTAL, D_MODEL), jnp.float32),
    mesh=plsc.VectorSubcoreMesh(core_axis_name="c", subcore_axis_name="s"),
    scratch_types=[
        pltpu.VMEM((N_CHUNKS, CHUNK), jnp.int32),
        pltpu.VMEM((NBUF, CHUNK, D_MODEL), jnp.float32),
    ] + [pltpu.SemaphoreType.DMA] * (2 * NBUF),
)
def _emb_call(table_hbm, x_hbm, out_hbm, idx_v, rows_v, *sems):
    _emb_kernel(table_hbm, x_hbm, out_hbm, idx_v, rows_v, *sems)


def kernel(x, table):
    xs = x.shape
    x_flat = x.astype(jnp.int32).reshape(NW, N_CHUNKS, CHUNK)
    out = _emb_call(table, x_flat)
    return out.reshape(xs[0], xs[1], D_MODEL)
